# Initial kernel scaffold; baseline (speedup 1.0000x reference)
import jax
import jax.numpy as jnp
from jax.experimental import pallas as pl


def kernel(x, tables):
    raise NotImplementedError("write your pallas kernel here")


# speedup vs baseline: 1.2469x; 1.2469x over previous
import functools

import jax
import jax.numpy as jnp
from jax import lax
from jax.experimental import pallas as pl
from jax.experimental.pallas import tpu as pltpu
from jax.experimental.pallas import tpu_sc as plsc

_NUM_FIELDS = 26
_VOCAB = 100000
_EMBED_DIM = 32
_BATCH = 16384
_N = _NUM_FIELDS * _BATCH
_NW = 32
_PER_W = _N // _NW
_CHUNK = 1024
_NCHUNK = _PER_W // _CHUNK
_LOG2_BATCH = 14


def _make_gather():
    mesh = plsc.VectorSubcoreMesh(core_axis_name="c", subcore_axis_name="s")

    @functools.partial(
        pl.kernel,
        mesh=mesh,
        out_type=jax.ShapeDtypeStruct((_N, _EMBED_DIM), jnp.float32),
        scratch_types=[
            pltpu.VMEM((_CHUNK,), jnp.int32),
            pltpu.VMEM((_CHUNK, _EMBED_DIM), jnp.float32),
            pltpu.SemaphoreType.DMA,
        ],
        compiler_params=pltpu.CompilerParams(use_tc_tiling_on_sc=False),
    )
    def gather_kernel(idx_hbm, tab_hbm, out_hbm, idx_v, rows_v, sem):
        wid = lax.axis_index("s") * 2 + lax.axis_index("c")
        w_base = wid * _PER_W

        def chunk_body(c, _):
            base = w_base + c * _CHUNK
            field = lax.shift_right_logical(base, _LOG2_BATCH)
            off = field * _VOCAB
            pltpu.sync_copy(idx_hbm.at[pl.ds(base, _CHUNK)], idx_v)

            def add_body(j, _):
                sl = pl.ds(j * 16, 16)
                idx_v[sl] = idx_v[sl] + jnp.full((16,), off, jnp.int32)
                return 0

            lax.fori_loop(0, _CHUNK // 16, add_body, 0)
            pltpu.async_copy(tab_hbm.at[idx_v], rows_v, sem).wait()
            pltpu.sync_copy(rows_v, out_hbm.at[pl.ds(base, _CHUNK)])
            return 0

        lax.fori_loop(0, _NCHUNK, chunk_body, 0)

    return gather_kernel


_gather = _make_gather()


def kernel(x, tables):
    idx_flat = x.T.reshape(_N)
    tab_flat = tables.reshape(_NUM_FIELDS * _VOCAB, _EMBED_DIM)
    out_flat = _gather(idx_flat, tab_flat)
    return out_flat.reshape(_NUM_FIELDS, _BATCH, _EMBED_DIM)


# speedup vs baseline: 4.6281x; 3.7116x over previous
import functools

import jax
import jax.numpy as jnp
from jax import lax
from jax.experimental import pallas as pl
from jax.experimental.pallas import tpu as pltpu
from jax.experimental.pallas import tpu_sc as plsc

_NUM_FIELDS = 26
_VOCAB = 100000
_EMBED_DIM = 32
_BATCH = 16384
_HALF = _BATCH // 2
_NW = 32
_TASKS_PER_W = _NUM_FIELDS * _EMBED_DIM // _NW


def _make_gather():
    mesh = plsc.VectorSubcoreMesh(core_axis_name="c", subcore_axis_name="s")

    @functools.partial(
        pl.kernel,
        mesh=mesh,
        out_type=jax.ShapeDtypeStruct((_NUM_FIELDS, _EMBED_DIM, _BATCH),
                                      jnp.float32),
        scratch_types=[
            pltpu.VMEM((_VOCAB,), jnp.float32),
            pltpu.VMEM((_HALF,), jnp.int32),
            pltpu.VMEM((_HALF,), jnp.float32),
        ],
        compiler_params=pltpu.CompilerParams(use_tc_tiling_on_sc=True,
                                             needs_layout_passes=False),
    )
    def gather_kernel(xt_hbm, tt_hbm, out_hbm, plane_v, xf_v, out_v):
        wid = lax.axis_index("s") * 2 + lax.axis_index("c")
        t0 = wid * _TASKS_PER_W

        def task_body(t, _):
            f = lax.shift_right_logical(t, 5)
            d = jnp.bitwise_and(t, 31)
            pltpu.sync_copy(tt_hbm.at[f, d], plane_v)
            for h in range(2):
                pltpu.sync_copy(xt_hbm.at[f, pl.ds(h * _HALF, _HALF)], xf_v)

                def g_body(g, _):
                    sl = pl.ds(g * 16, 16)
                    out_v[sl] = plsc.load_gather(plane_v, [xf_v[sl]])
                    return 0

                lax.fori_loop(0, _HALF // 16, g_body, 0)
                pltpu.sync_copy(out_v, out_hbm.at[f, d, pl.ds(h * _HALF, _HALF)])
            return 0

        lax.fori_loop(t0, t0 + _TASKS_PER_W, task_body, 0)

    return gather_kernel


_gather = _make_gather()


def kernel(x, tables):
    out = _gather(x.T, jnp.transpose(tables, (0, 2, 1)))
    return jnp.transpose(out, (0, 2, 1))


# speedup vs baseline: 6.9594x; 1.5037x over previous
import functools

import jax
import jax.numpy as jnp
from jax import lax
from jax.experimental import pallas as pl
from jax.experimental.pallas import tpu as pltpu
from jax.experimental.pallas import tpu_sc as plsc

_NUM_FIELDS = 26
_VOCAB = 100000
_EMBED_DIM = 32
_BATCH = 16384
_HALF = _BATCH // 2
_NW = 32
_TASKS_PER_W = _NUM_FIELDS * _EMBED_DIM // _NW


def _make_gather():
    mesh = plsc.VectorSubcoreMesh(core_axis_name="c", subcore_axis_name="s")

    @functools.partial(
        pl.kernel,
        mesh=mesh,
        out_type=jax.ShapeDtypeStruct((_NUM_FIELDS, _EMBED_DIM, _BATCH),
                                      jnp.float32),
        scratch_types=[
            pltpu.VMEM((_VOCAB,), jnp.float32),
            pltpu.VMEM((_HALF,), jnp.int32),
            pltpu.VMEM((_HALF,), jnp.float32),
        ],
        compiler_params=pltpu.CompilerParams(use_tc_tiling_on_sc=True,
                                             needs_layout_passes=False),
    )
    def gather_kernel(xt_hbm, tt_hbm, out_hbm, plane_v, xf_v, out_v):
        wid = lax.axis_index("s") * 2 + lax.axis_index("c")
        t0 = wid * _TASKS_PER_W

        def task_body(t, _):
            f = lax.shift_right_logical(t, 5)
            d = jnp.bitwise_and(t, 31)
            pltpu.sync_copy(tt_hbm.at[f, d], plane_v)
            for h in range(2):
                pltpu.sync_copy(xt_hbm.at[f, pl.ds(h * _HALF, _HALF)], xf_v)

                @plsc.parallel_loop(0, _HALF, step=16, unroll=8)
                def g_body(b):
                    sl = pl.ds(b, 16)
                    out_v[sl] = plsc.load_gather(plane_v, [xf_v[sl]])
                pltpu.sync_copy(out_v, out_hbm.at[f, d, pl.ds(h * _HALF, _HALF)])
            return 0

        lax.fori_loop(t0, t0 + _TASKS_PER_W, task_body, 0)

    return gather_kernel


_gather = _make_gather()


def kernel(x, tables):
    out = _gather(x.T, jnp.transpose(tables, (0, 2, 1)))
    return jnp.transpose(out, (0, 2, 1))


# speedup vs baseline: 9.6057x; 1.3803x over previous
import functools

import jax
import jax.numpy as jnp
from jax import lax
from jax.experimental import pallas as pl
from jax.experimental.pallas import tpu as pltpu
from jax.experimental.pallas import tpu_sc as plsc

_NUM_FIELDS = 26
_VOCAB = 100000
_EMBED_DIM = 32
_BATCH = 16384
_Q = _BATCH // 4
_NW = 32
_TASKS_PER_W = _NUM_FIELDS * _EMBED_DIM // _NW


def _make_gather():
    mesh = plsc.VectorSubcoreMesh(core_axis_name="c", subcore_axis_name="s")

    @functools.partial(
        pl.kernel,
        mesh=mesh,
        out_type=jax.ShapeDtypeStruct((_NUM_FIELDS, _EMBED_DIM, _BATCH),
                                      jnp.float32),
        scratch_types=[
            pltpu.VMEM((_VOCAB,), jnp.float32),
            pltpu.VMEM((_BATCH,), jnp.int32),
            pltpu.VMEM((_Q,), jnp.float32),
            pltpu.VMEM((_Q,), jnp.float32),
            pltpu.SemaphoreType.DMA,
            pltpu.SemaphoreType.DMA,
            pltpu.SemaphoreType.DMA,
        ],
        compiler_params=pltpu.CompilerParams(use_tc_tiling_on_sc=True,
                                             needs_layout_passes=False),
    )
    def gather_kernel(xt_hbm, tt_hbm, out_hbm, plane_v, xf_v, out_v0, out_v1,
                      psem, osem0, osem1):
        wid = lax.axis_index("s") * 2 + lax.axis_index("c")
        t0 = wid * _TASKS_PER_W
        outs = (out_v0, out_v1)
        osems = (osem0, osem1)

        def task_body(t, prev_f):
            f = lax.shift_right_logical(t, 5)
            d = jnp.bitwise_and(t, 31)
            plane_cp = pltpu.async_copy(tt_hbm.at[f, d], plane_v, psem)

            @pl.when(f != prev_f)
            def _load_indices():
                pltpu.sync_copy(xt_hbm.at[f], xf_v)

            plane_cp.wait()
            for q in range(4):
                ov = outs[q & 1]
                osem = osems[q & 1]
                if q >= 2:
                    pltpu.make_async_copy(
                        ov, out_hbm.at[f, d, pl.ds((q - 2) * _Q, _Q)], osem
                    ).wait()
                else:
                    @pl.when(t > t0)
                    def _drain_prev_task():
                        pltpu.make_async_copy(
                            ov, out_hbm.at[f, d, pl.ds(q * _Q, _Q)], osem
                        ).wait()

                @plsc.parallel_loop(0, _Q, step=16, unroll=8)
                def g_body(b):
                    ov[pl.ds(b, 16)] = plsc.load_gather(
                        plane_v, [xf_v[pl.ds(q * _Q + b, 16)]])

                pltpu.async_copy(ov, out_hbm.at[f, d, pl.ds(q * _Q, _Q)], osem)
            return f

        lax.fori_loop(t0, t0 + _TASKS_PER_W, task_body, jnp.int32(-1))

        t_last = t0 + _TASKS_PER_W - 1
        f_last = lax.shift_right_logical(t_last, 5)
        d_last = jnp.bitwise_and(t_last, 31)
        for q in (2, 3):
            pltpu.make_async_copy(
                outs[q & 1], out_hbm.at[f_last, d_last, pl.ds(q * _Q, _Q)],
                osems[q & 1]
            ).wait()

    return gather_kernel


_gather = _make_gather()


def kernel(x, tables):
    out = _gather(x.T, jnp.transpose(tables, (0, 2, 1)))
    return jnp.transpose(out, (0, 2, 1))


# speedup vs baseline: 9.6306x; 1.0026x over previous
import functools

import jax
import jax.numpy as jnp
from jax import lax
from jax.experimental import pallas as pl
from jax.experimental.pallas import tpu as pltpu
from jax.experimental.pallas import tpu_sc as plsc

_NUM_FIELDS = 26
_VOCAB = 100000
_EMBED_DIM = 32
_BATCH = 16384
_Q = _BATCH // 4
_NW = 32
_TASKS_PER_W = _NUM_FIELDS * _EMBED_DIM // _NW


def _make_gather():
    mesh = plsc.VectorSubcoreMesh(core_axis_name="c", subcore_axis_name="s")

    @functools.partial(
        pl.kernel,
        mesh=mesh,
        out_type=jax.ShapeDtypeStruct((_NUM_FIELDS, _EMBED_DIM, _BATCH),
                                      jnp.float32),
        scratch_types=[
            pltpu.VMEM((_VOCAB,), jnp.float32),
            pltpu.VMEM((_BATCH,), jnp.int32),
            pltpu.VMEM((_Q,), jnp.float32),
            pltpu.VMEM((_Q,), jnp.float32),
            pltpu.SemaphoreType.DMA,
            pltpu.SemaphoreType.DMA,
            pltpu.SemaphoreType.DMA,
        ],
        compiler_params=pltpu.CompilerParams(use_tc_tiling_on_sc=True,
                                             needs_layout_passes=False),
    )
    def gather_kernel(xt_hbm, tt_hbm, out_hbm, plane_v, xf_v, out_v0, out_v1,
                      psem, osem0, osem1):
        wid = lax.axis_index("s") * 2 + lax.axis_index("c")
        t0 = wid * _TASKS_PER_W
        outs = (out_v0, out_v1)
        osems = (osem0, osem1)

        def task_body(t, prev_f):
            f = lax.shift_right_logical(t, 5)
            d = jnp.bitwise_and(t, 31)
            plane_cp = pltpu.async_copy(tt_hbm.at[f, d], plane_v, psem)

            @pl.when(f != prev_f)
            def _load_indices():
                pltpu.sync_copy(xt_hbm.at[f], xf_v)

            plane_cp.wait()
            for q in range(4):
                ov = outs[q & 1]
                osem = osems[q & 1]
                if q >= 2:
                    pltpu.make_async_copy(
                        ov, out_hbm.at[f, d, pl.ds((q - 2) * _Q, _Q)], osem
                    ).wait()
                else:
                    @pl.when(t > t0)
                    def _drain_prev_task():
                        pltpu.make_async_copy(
                            ov, out_hbm.at[f, d, pl.ds(q * _Q, _Q)], osem
                        ).wait()

                @plsc.parallel_loop(0, _Q, step=16, unroll=16)
                def g_body(b):
                    ov[pl.ds(b, 16)] = plsc.load_gather(
                        plane_v, [xf_v[pl.ds(q * _Q + b, 16)]])

                pltpu.async_copy(ov, out_hbm.at[f, d, pl.ds(q * _Q, _Q)], osem)
            return f

        lax.fori_loop(t0, t0 + _TASKS_PER_W, task_body, jnp.int32(-1))

        t_last = t0 + _TASKS_PER_W - 1
        f_last = lax.shift_right_logical(t_last, 5)
        d_last = jnp.bitwise_and(t_last, 31)
        for q in (2, 3):
            pltpu.make_async_copy(
                outs[q & 1], out_hbm.at[f_last, d_last, pl.ds(q * _Q, _Q)],
                osems[q & 1]
            ).wait()

    return gather_kernel


_gather = _make_gather()


def kernel(x, tables):
    out = _gather(x.T, jnp.transpose(tables, (0, 2, 1)))
    return jnp.transpose(out, (0, 2, 1))


# speedup vs baseline: 9.6384x; 1.0008x over previous
import functools

import jax
import jax.numpy as jnp
from jax import lax
from jax.experimental import pallas as pl
from jax.experimental.pallas import tpu as pltpu
from jax.experimental.pallas import tpu_sc as plsc

_NUM_FIELDS = 26
_VOCAB = 100000
_EMBED_DIM = 32
_BATCH = 16384
_Q = _BATCH // 4
_NW = 32
_TASKS_PER_W = _NUM_FIELDS * _EMBED_DIM // _NW


@functools.cache
def _make_gather():
    mesh = plsc.VectorSubcoreMesh(core_axis_name="c", subcore_axis_name="s")

    @functools.partial(
        pl.kernel,
        mesh=mesh,
        out_type=jax.ShapeDtypeStruct((_NUM_FIELDS, _EMBED_DIM, _BATCH),
                                      jnp.float32),
        scratch_types=[
            pltpu.VMEM((_VOCAB,), jnp.float32),
            pltpu.VMEM((_BATCH,), jnp.int32),
            pltpu.VMEM((_Q,), jnp.float32),
            pltpu.VMEM((_Q,), jnp.float32),
            pltpu.SemaphoreType.DMA,
            pltpu.SemaphoreType.DMA,
            pltpu.SemaphoreType.DMA,
        ],
        compiler_params=pltpu.CompilerParams(use_tc_tiling_on_sc=True,
                                             needs_layout_passes=False),
    )
    def gather_kernel(xt_hbm, tt_hbm, out_hbm, plane_v, xf_v, out_v0, out_v1,
                      psem, osem0, osem1):
        wid = lax.axis_index("s") * 2 + lax.axis_index("c")
        t0 = wid * _TASKS_PER_W
        outs = (out_v0, out_v1)
        osems = (osem0, osem1)

        def task_body(t, prev_f):
            f = lax.shift_right_logical(t, 5)
            d = jnp.bitwise_and(t, 31)
            plane_cp = pltpu.async_copy(tt_hbm.at[f, d], plane_v, psem)

            @pl.when(f != prev_f)
            def _load_indices():
                pltpu.sync_copy(xt_hbm.at[f], xf_v)

            plane_cp.wait()
            for q in range(4):
                ov = outs[q & 1]
                osem = osems[q & 1]
                if q >= 2:
                    pltpu.make_async_copy(
                        ov, out_hbm.at[f, d, pl.ds((q - 2) * _Q, _Q)], osem
                    ).wait()
                else:
                    @pl.when(t > t0)
                    def _drain_prev_task():
                        pltpu.make_async_copy(
                            ov, out_hbm.at[f, d, pl.ds(q * _Q, _Q)], osem
                        ).wait()

                @plsc.parallel_loop(0, _Q, step=16, unroll=16)
                def g_body(b):
                    ov[pl.ds(b, 16)] = plsc.load_gather(
                        plane_v, [xf_v[pl.ds(q * _Q + b, 16)]])

                pltpu.async_copy(ov, out_hbm.at[f, d, pl.ds(q * _Q, _Q)], osem)
            return f

        lax.fori_loop(t0, t0 + _TASKS_PER_W, task_body, jnp.int32(-1))

        t_last = t0 + _TASKS_PER_W - 1
        f_last = lax.shift_right_logical(t_last, 5)
        d_last = jnp.bitwise_and(t_last, 31)
        for q in (2, 3):
            pltpu.make_async_copy(
                outs[q & 1], out_hbm.at[f_last, d_last, pl.ds(q * _Q, _Q)],
                osems[q & 1]
            ).wait()

    return gather_kernel


def kernel(x, tables):
    out = _make_gather()(x.T, jnp.transpose(tables, (0, 2, 1)))
    return jnp.transpose(out, (0, 2, 1))
